# baseline (speedup 1.0000x reference)
import functools

import jax
import jax.numpy as jnp
from jax import lax
from jax.experimental import pallas as pl
from jax.experimental.pallas import tpu as pltpu
from jax.experimental.pallas import tpu_sc as plsc

SHAPE = (16384, 128)
T_USED = 10

_N = SHAPE[0] * SHAPE[1]
_NC = 2
_NS = 16
_NW = _NC * _NS
_PER_W = _N // _NW
_CH = 32768
_NSUB = _PER_W // _CH
_LANES = 16

_MESH = plsc.VectorSubcoreMesh(core_axis_name="c", subcore_axis_name="s")


@functools.partial(
    pl.kernel,
    mesh=_MESH,
    out_type=jax.ShapeDtypeStruct((_N,), jnp.float32),
    scratch_types=[
        pltpu.VMEM((_CH,), jnp.float32),
        pltpu.VMEM((_CH,), jnp.float32),
        pltpu.VMEM((_CH,), jnp.float32),
        pltpu.SemaphoreType.DMA,
        pltpu.SemaphoreType.DMA,
    ],
)
def _sc_sum(ap_hbm, hist_hbm, out_hbm, acc, bufa, bufb, sema, semb):
    wid = lax.axis_index("s") * _NC + lax.axis_index("c")
    base = wid * _PER_W
    bufs = (bufa, bufb)
    sems = (sema, semb)
    for s in range(_NSUB):
        off = base + s * _CH
        copies = [None] * T_USED
        copies[0] = pltpu.async_copy(hist_hbm.at[0, pl.ds(off, _CH)], bufa, sema)
        pltpu.sync_copy(ap_hbm.at[pl.ds(off, _CH)], acc)
        for t in range(T_USED):
            if t + 1 < T_USED:
                copies[t + 1] = pltpu.async_copy(
                    hist_hbm.at[t + 1, pl.ds(off, _CH)],
                    bufs[(t + 1) % 2],
                    sems[(t + 1) % 2],
                )
            copies[t].wait()
            buf = bufs[t % 2]

            def body(g, _, buf=buf):
                sl = pl.ds(g * _LANES, _LANES)
                plsc.addupdate(acc.at[sl], buf[sl])
                return 0

            lax.fori_loop(0, _CH // _LANES, body, 0)
        pltpu.sync_copy(acc, out_hbm.at[pl.ds(off, _CH)])


def kernel(action_potential, action_potential_history):
    ap_flat = action_potential.reshape(_N)
    hist_flat = action_potential_history.reshape(
        action_potential_history.shape[0], _N
    )
    out = _sc_sum(ap_flat, hist_flat)
    return out.reshape(SHAPE)


# speedup vs baseline: 4.6104x; 4.6104x over previous
import functools

import jax
import jax.numpy as jnp
from jax import lax
from jax.experimental import pallas as pl
from jax.experimental.pallas import tpu as pltpu
from jax.experimental.pallas import tpu_sc as plsc

SHAPE = (16384, 128)
T_USED = 10

_NC = 2
_NS = 16
_NW = _NC * _NS
_ROWS_W = SHAPE[0] // _NW
_CH_ROWS = 256
_NSUB = _ROWS_W // _CH_ROWS
_LANES = 16
_GPR = SHAPE[1] // _LANES

_MESH = plsc.VectorSubcoreMesh(core_axis_name="c", subcore_axis_name="s")


@functools.partial(
    pl.kernel,
    mesh=_MESH,
    out_type=jax.ShapeDtypeStruct(SHAPE, jnp.float32),
    scratch_types=[
        pltpu.VMEM((_CH_ROWS, SHAPE[1]), jnp.float32),
        pltpu.VMEM((_CH_ROWS, SHAPE[1]), jnp.float32),
        pltpu.VMEM((_CH_ROWS, SHAPE[1]), jnp.float32),
        pltpu.SemaphoreType.DMA,
        pltpu.SemaphoreType.DMA,
    ],
)
def _sc_sum(ap_hbm, hist_hbm, out_hbm, acc, bufa, bufb, sema, semb):
    wid = lax.axis_index("s") * _NC + lax.axis_index("c")
    base = wid * _ROWS_W
    bufs = (bufa, bufb)
    sems = (sema, semb)
    for s in range(_NSUB):
        r0 = base + s * _CH_ROWS
        copies = [None] * T_USED
        copies[0] = pltpu.async_copy(
            hist_hbm.at[0, pl.ds(r0, _CH_ROWS), :], bufa, sema
        )
        pltpu.sync_copy(ap_hbm.at[pl.ds(r0, _CH_ROWS), :], acc)
        for t in range(T_USED):
            if t + 1 < T_USED:
                copies[t + 1] = pltpu.async_copy(
                    hist_hbm.at[t + 1, pl.ds(r0, _CH_ROWS), :],
                    bufs[(t + 1) % 2],
                    sems[(t + 1) % 2],
                )
            copies[t].wait()
            buf = bufs[t % 2]

            def body(r, _, buf=buf):
                for g in range(_GPR):
                    sl = pl.ds(g * _LANES, _LANES)
                    plsc.addupdate(acc.at[r, sl], buf[r, sl])
                return 0

            lax.fori_loop(0, _CH_ROWS, body, 0)
        pltpu.sync_copy(acc, out_hbm.at[pl.ds(r0, _CH_ROWS), :])


def kernel(action_potential, action_potential_history):
    return _sc_sum(action_potential, action_potential_history)
